# baseline (speedup 1.0000x reference)
import functools

import jax
import jax.numpy as jnp
from jax import lax
from jax.experimental import pallas as pl
from jax.experimental.pallas import tpu as pltpu
from jax.experimental.pallas import tpu_sc as plsc

N_ROWS = 16384
N_COLS = 256
K_OUT = 128
NUM_WORKERS = 32
ROWS_PER_WORKER = N_ROWS // NUM_WORKERS
BLK_ROWS = 64
NBLK = ROWS_PER_WORKER // BLK_ROWS
LANES = 16


def _build_sc_kernel():
    mesh = plsc.VectorSubcoreMesh(core_axis_name="c", subcore_axis_name="s")

    @functools.partial(
        pl.kernel,
        mesh=mesh,
        out_type=jax.ShapeDtypeStruct((N_ROWS, K_OUT), jnp.float32),
        compiler_params=pltpu.CompilerParams(
            needs_layout_passes=False,
            use_tc_tiling_on_sc=True,
        ),
        scratch_types=[
            pltpu.VMEM((2, BLK_ROWS, N_COLS), jnp.float32),
            pltpu.VMEM((2, BLK_ROWS, K_OUT), jnp.float32),
            pltpu.SemaphoreType.DMA((2,)),
            pltpu.SemaphoreType.DMA((2,)),
        ],
    )
    def k(in_hbm, out_hbm, in_v, out_v, in_sem, out_sem):
        wid = lax.axis_index("s") * 2 + lax.axis_index("c")
        lane2 = 2 * lax.iota(jnp.int32, LANES)
        cols = [lane2 + (2 * LANES * t) for t in range(K_OUT // LANES)]

        def row0(b):
            return wid * ROWS_PER_WORKER + b * BLK_ROWS

        def in_copy(b):
            par = lax.rem(b, 2)
            return pltpu.make_async_copy(
                in_hbm.at[pl.ds(row0(b), BLK_ROWS), :],
                in_v.at[par],
                in_sem.at[par],
            )

        def out_copy(b):
            par = lax.rem(b, 2)
            return pltpu.make_async_copy(
                out_v.at[par],
                out_hbm.at[pl.ds(row0(b), BLK_ROWS), :],
                out_sem.at[par],
            )

        def compute(b):
            par = lax.rem(b, 2)
            src = in_v.at[par]
            dst = out_v.at[par]

            @plsc.parallel_loop(0, BLK_ROWS, unroll=2)
            def body(r):
                rows = jnp.full((LANES,), r, jnp.int32)
                for t in range(K_OUT // LANES):
                    v = plsc.load_gather(src, [rows, cols[t]])
                    dst[r, pl.ds(LANES * t, LANES)] = v

        in_copy(0).start()
        for b in range(2):
            in_copy(b + 1).start()
            in_copy(b).wait()
            compute(b)
            out_copy(b).start()

        def block(b, _):
            in_copy(b + 1).start()
            in_copy(b).wait()
            out_copy(b - 2).wait()
            compute(b)
            out_copy(b).start()
            return 0

        lax.fori_loop(2, NBLK - 1, block, 0)

        b_last = NBLK - 1
        in_copy(b_last).wait()
        out_copy(b_last - 2).wait()
        compute(b_last)
        out_copy(b_last).start()
        out_copy(NBLK - 2).wait()
        out_copy(NBLK - 1).wait()

    return k


_SC_KERNEL = _build_sc_kernel()


def kernel(inputs):
    return _SC_KERNEL(inputs)


# speedup vs baseline: 1.0004x; 1.0004x over previous
import functools

import jax
import jax.numpy as jnp
from jax import lax
from jax.experimental import pallas as pl
from jax.experimental.pallas import tpu as pltpu
from jax.experimental.pallas import tpu_sc as plsc

N_ROWS = 16384
N_COLS = 256
K_OUT = 128
NUM_WORKERS = 32
ROWS_PER_WORKER = N_ROWS // NUM_WORKERS
BLK_ROWS = 128
NBLK = ROWS_PER_WORKER // BLK_ROWS
LANES = 16


def _build_sc_kernel():
    mesh = plsc.VectorSubcoreMesh(core_axis_name="c", subcore_axis_name="s")

    @functools.partial(
        pl.kernel,
        mesh=mesh,
        out_type=jax.ShapeDtypeStruct((N_ROWS, K_OUT), jnp.float32),
        compiler_params=pltpu.CompilerParams(
            needs_layout_passes=False,
            use_tc_tiling_on_sc=True,
        ),
        scratch_types=[
            pltpu.VMEM((2, BLK_ROWS, N_COLS), jnp.float32),
            pltpu.VMEM((2, BLK_ROWS, K_OUT), jnp.float32),
            pltpu.SemaphoreType.DMA((2,)),
            pltpu.SemaphoreType.DMA((2,)),
        ],
    )
    def k(in_hbm, out_hbm, in_v, out_v, in_sem, out_sem):
        wid = lax.axis_index("s") * 2 + lax.axis_index("c")
        lane2 = 2 * lax.iota(jnp.int32, LANES)
        cols = [lane2 + (2 * LANES * t) for t in range(K_OUT // LANES)]

        def row0(b):
            return wid * ROWS_PER_WORKER + b * BLK_ROWS

        def in_copy(b):
            par = lax.rem(b, 2)
            return pltpu.make_async_copy(
                in_hbm.at[pl.ds(row0(b), BLK_ROWS), :],
                in_v.at[par],
                in_sem.at[par],
            )

        def out_copy(b):
            par = lax.rem(b, 2)
            return pltpu.make_async_copy(
                out_v.at[par],
                out_hbm.at[pl.ds(row0(b), BLK_ROWS), :],
                out_sem.at[par],
            )

        def compute(b):
            par = lax.rem(b, 2)
            src = in_v.at[par]
            dst = out_v.at[par]

            @plsc.parallel_loop(0, BLK_ROWS, unroll=2)
            def body(r):
                rows = jnp.full((LANES,), r, jnp.int32)
                for t in range(K_OUT // LANES):
                    v = plsc.load_gather(src, [rows, cols[t]])
                    dst[r, pl.ds(LANES * t, LANES)] = v

        in_copy(0).start()
        for b in range(2):
            in_copy(b + 1).start()
            in_copy(b).wait()
            compute(b)
            out_copy(b).start()

        def block(b, _):
            in_copy(b + 1).start()
            in_copy(b).wait()
            out_copy(b - 2).wait()
            compute(b)
            out_copy(b).start()
            return 0

        lax.fori_loop(2, NBLK - 1, block, 0)

        b_last = NBLK - 1
        in_copy(b_last).wait()
        out_copy(b_last - 2).wait()
        compute(b_last)
        out_copy(b_last).start()
        out_copy(NBLK - 2).wait()
        out_copy(NBLK - 1).wait()

    return k


_SC_KERNEL = _build_sc_kernel()


def kernel(inputs):
    return _SC_KERNEL(inputs)
